# initial kernel scaffold (unmeasured)
import jax
import jax.numpy as jnp
from jax import lax
from jax.experimental import pallas as pl
from jax.experimental.pallas import tpu as pltpu

N_DEV = 8
M_PER = 512
N = 8192
NH = 4096
F32 = jnp.float32
BF16 = jnp.bfloat16


def _ring_to_logical(ring_pos):
    return jnp.where(ring_pos < 4, ring_pos, 11 - ring_pos)


def kernel(x, w_mat):
    assert x.shape == (4096, 512) and w_mat.shape == (512, N), (
        x.shape,
        w_mat.shape,
    )

    def body(
        x_ref,
        w_ref,
        out_ref,
        buf_r,
        buf_l,
        ssem_r,
        rsem_r,
        ssem_l,
        rsem_l,
        my_amax,
        rx_amax,
        ssem_ax,
        rsem_ax,
    ):
        my_log = lax.axis_index("i")
        pos = _ring_to_logical(my_log)
        right_log = _ring_to_logical((pos + 1) % N_DEV)
        left_log = _ring_to_logical((pos - 1) % N_DEV)

        barrier_sem = pltpu.get_barrier_semaphore()
        for nbr in (left_log, right_log):
            pl.semaphore_signal(
                barrier_sem,
                inc=1,
                device_id=(nbr,),
                device_id_type=pl.DeviceIdType.MESH,
            )
        pl.semaphore_wait(barrier_sem, 2)

        def partial(chunk_log, lo, hi):
            return jnp.dot(
                x_ref[pl.ds(chunk_log * M_PER, M_PER), :],
                w_ref[:, lo:hi],
                preferred_element_type=F32,
            )

        rdma_r = [None] * (N_DEV - 1)
        rdma_l = [None] * (N_DEV - 1)
        for s in range(N_DEV - 1):
            c_r = _ring_to_logical((pos - 1 - s) % N_DEV)
            c_l = _ring_to_logical((pos + 1 + s) % N_DEV)
            p_r = partial(c_r, 0, NH)
            p_l = partial(c_l, NH, N)

            if s == 0:
                buf_r[N_DEV - 1] = p_r.astype(BF16)
                buf_l[N_DEV - 1] = p_l.astype(BF16)
                src = N_DEV - 1
            else:
                rdma_r[s - 1].wait_recv()
                buf_r[s - 1] = (buf_r[s - 1].astype(F32) + p_r).astype(BF16)
                rdma_l[s - 1].wait_recv()
                buf_l[s - 1] = (buf_l[s - 1].astype(F32) + p_l).astype(BF16)
                src = s - 1

            rdma_r[s] = pltpu.make_async_remote_copy(
                src_ref=buf_r.at[src],
                dst_ref=buf_r.at[s],
                send_sem=ssem_r.at[s],
                recv_sem=rsem_r.at[s],
                device_id=(right_log,),
                device_id_type=pl.DeviceIdType.MESH,
            )
            rdma_r[s].start()
            rdma_l[s] = pltpu.make_async_remote_copy(
                src_ref=buf_l.at[src],
                dst_ref=buf_l.at[s],
                send_sem=ssem_l.at[s],
                recv_sem=rsem_l.at[s],
                device_id=(left_log,),
                device_id_type=pl.DeviceIdType.MESH,
            )
            rdma_l[s].start()

            if s == 0:
                out_ref[:, 0:NH] = partial(my_log, 0, NH)
                out_ref[:, NH:N] = partial(my_log, NH, N)

        rdma_r[N_DEV - 2].wait_recv()
        out_ref[:, 0:NH] = jnp.maximum(
            out_ref[:, 0:NH] + buf_r[N_DEV - 2].astype(F32), 0.0
        )
        rdma_l[N_DEV - 2].wait_recv()
        out_ref[:, NH:N] = jnp.maximum(
            out_ref[:, NH:N] + buf_l[N_DEV - 2].astype(F32), 0.0
        )

        m_in = my_log % 4
        base = my_log - m_in
        partners = [
            base + jnp.bitwise_xor(m_in, 1),
            base + (3 - m_in),
            (my_log + 4) % N_DEV,
        ]
        my_amax[0, :] = jnp.full((128,), jnp.max(out_ref[:, :]), F32)
        for h, pid in enumerate(partners):
            ex = pltpu.make_async_remote_copy(
                src_ref=my_amax,
                dst_ref=rx_amax.at[h],
                send_sem=ssem_ax.at[h],
                recv_sem=rsem_ax.at[h],
                device_id=(pid,),
                device_id_type=pl.DeviceIdType.MESH,
            )
            ex.start()
            ex.wait()
            my_amax[0, :] = jnp.maximum(my_amax[0, :], rx_amax[h, 0, :])

        scale = my_amax[0, 0] / 127.0
        for lo, hi in ((0, NH), (NH, N)):
            q = jnp.clip(jnp.round(out_ref[:, lo:hi] / scale), -127.0, 127.0)
            out_ref[:, lo:hi] = q * scale

        for s in range(N_DEV - 1):
            rdma_r[s].wait_send()
            rdma_l[s].wait_send()

    return pl.pallas_call(
        body,
        out_shape=jax.ShapeDtypeStruct((M_PER, N), F32),
        in_specs=[
            pl.BlockSpec(memory_space=pltpu.VMEM),
            pl.BlockSpec(memory_space=pltpu.VMEM),
        ],
        out_specs=pl.BlockSpec(memory_space=pltpu.VMEM),
        scratch_shapes=[
            pltpu.VMEM((N_DEV, M_PER, NH), BF16),
            pltpu.VMEM((N_DEV, M_PER, NH), BF16),
            pltpu.SemaphoreType.DMA((N_DEV - 1,)),
            pltpu.SemaphoreType.DMA((N_DEV - 1,)),
            pltpu.SemaphoreType.DMA((N_DEV - 1,)),
            pltpu.SemaphoreType.DMA((N_DEV - 1,)),
            pltpu.VMEM((1, 128), F32),
            pltpu.VMEM((3, 1, 128), F32),
            pltpu.SemaphoreType.DMA((3,)),
            pltpu.SemaphoreType.DMA((3,)),
        ],
        compiler_params=pltpu.CompilerParams(collective_id=0),
    )(x, w_mat)


# baseline (device time: 429174 ns/iter reference)
import jax
import jax.numpy as jnp
from jax import lax
from jax.experimental import pallas as pl
from jax.experimental.pallas import tpu as pltpu

N_DEV = 8
M_PER = 512
N = 8192
NH = 4096
CB = 512
F32 = jnp.float32
BF16 = jnp.bfloat16


def _ring_to_logical(ring_pos):
    return jnp.where(ring_pos < 4, ring_pos, 11 - ring_pos)


def kernel(x, w_mat):
    assert x.shape == (4096, 512) and w_mat.shape == (512, N)
    x = x.astype(BF16)
    w_mat = w_mat.astype(BF16)

    def body(
        x_ref,
        w_ref,
        out_ref,
        buf_r,
        buf_l,
        ssem_r,
        rsem_r,
        ssem_l,
        rsem_l,
        cred_r,
        cred_l,
        my_amax,
        rx_amax,
        ssem_ax,
        rsem_ax,
    ):
        my_log = lax.axis_index("i")
        pos = _ring_to_logical(my_log)
        right_log = _ring_to_logical((pos + 1) % N_DEV)
        left_log = _ring_to_logical((pos - 1) % N_DEV)

        barrier_sem = pltpu.get_barrier_semaphore()
        for nbr in (left_log, right_log):
            pl.semaphore_signal(
                barrier_sem,
                inc=1,
                device_id=(nbr,),
                device_id_type=pl.DeviceIdType.MESH,
            )
        pl.semaphore_wait(barrier_sem, 2)

        def dot_block(chunk_log, lo, hi):
            return jnp.dot(
                x_ref[pl.ds(chunk_log * M_PER, M_PER), :],
                w_ref[:, lo:hi],
                preferred_element_type=F32,
            )

        def seed_slot(buf, chunk_log, base):
            for b in range(0, NH, CB):
                buf[1, :, b : b + CB] = dot_block(
                    chunk_log, base + b, base + b + CB
                ).astype(BF16)

        def accum_slot(buf, slot, chunk_log, base):
            for b in range(0, NH, CB):
                buf[slot, :, b : b + CB] = (
                    buf[slot, :, b : b + CB].astype(F32)
                    + dot_block(chunk_log, base + b, base + b + CB)
                ).astype(BF16)

        rdma_r = [None] * (N_DEV - 1)
        rdma_l = [None] * (N_DEV - 1)
        for s in range(N_DEV - 1):
            c_r = _ring_to_logical((pos - 1 - s) % N_DEV)
            c_l = _ring_to_logical((pos + 1 + s) % N_DEV)

            if s == 0:
                seed_slot(buf_r, c_r, 0)
                seed_slot(buf_l, c_l, NH)
            else:
                rdma_r[s - 1].wait_recv()
                accum_slot(buf_r, (s - 1) % 2, c_r, 0)
                rdma_l[s - 1].wait_recv()
                accum_slot(buf_l, (s - 1) % 2, c_l, NH)
                rdma_r[s - 1].wait_send()
                rdma_l[s - 1].wait_send()
                pl.semaphore_signal(
                    cred_r,
                    inc=1,
                    device_id=(left_log,),
                    device_id_type=pl.DeviceIdType.MESH,
                )
                pl.semaphore_signal(
                    cred_l,
                    inc=1,
                    device_id=(right_log,),
                    device_id_type=pl.DeviceIdType.MESH,
                )
                pl.semaphore_wait(cred_r, 1)
                pl.semaphore_wait(cred_l, 1)
            src = (s - 1) % 2

            rdma_r[s] = pltpu.make_async_remote_copy(
                src_ref=buf_r.at[src],
                dst_ref=buf_r.at[s % 2],
                send_sem=ssem_r.at[s % 2],
                recv_sem=rsem_r.at[s % 2],
                device_id=(right_log,),
                device_id_type=pl.DeviceIdType.MESH,
            )
            rdma_r[s].start()
            rdma_l[s] = pltpu.make_async_remote_copy(
                src_ref=buf_l.at[src],
                dst_ref=buf_l.at[s % 2],
                send_sem=ssem_l.at[s % 2],
                recv_sem=rsem_l.at[s % 2],
                device_id=(left_log,),
                device_id_type=pl.DeviceIdType.MESH,
            )
            rdma_l[s].start()

            if s == 0:
                for b in range(0, N, CB):
                    out_ref[:, b : b + CB] = dot_block(my_log, b, b + CB)

        rdma_r[N_DEV - 2].wait_recv()
        for b in range(0, NH, CB):
            out_ref[:, b : b + CB] = jnp.maximum(
                out_ref[:, b : b + CB] + buf_r[0, :, b : b + CB].astype(F32),
                0.0,
            )
        rdma_l[N_DEV - 2].wait_recv()
        for b in range(NH, N, CB):
            out_ref[:, b : b + CB] = jnp.maximum(
                out_ref[:, b : b + CB]
                + buf_l[0, :, b - NH : b - NH + CB].astype(F32),
                0.0,
            )
        rdma_r[N_DEV - 2].wait_send()
        rdma_l[N_DEV - 2].wait_send()

        m_in = my_log % 4
        base = my_log - m_in
        partners = [
            base + jnp.bitwise_xor(m_in, 1),
            base + (3 - m_in),
            (my_log + 4) % N_DEV,
        ]
        my_amax[:, :] = jnp.full((1, 128), jnp.max(out_ref[:, :]), F32)
        for h, pid in enumerate(partners):
            ex = pltpu.make_async_remote_copy(
                src_ref=my_amax,
                dst_ref=rx_amax.at[h],
                send_sem=ssem_ax.at[h],
                recv_sem=rsem_ax.at[h],
                device_id=(pid,),
                device_id_type=pl.DeviceIdType.MESH,
            )
            ex.start()
            ex.wait()
            my_amax[:, :] = jnp.maximum(my_amax[:, :], rx_amax[h, :, :])

        scale = my_amax[0, 0] / 127.0
        for b in range(0, N, CB):
            q = jnp.clip(
                jnp.round(out_ref[:, b : b + CB] / scale), -127.0, 127.0
            )
            out_ref[:, b : b + CB] = q * scale

    return pl.pallas_call(
        body,
        out_shape=jax.ShapeDtypeStruct((M_PER, N), F32),
        in_specs=[
            pl.BlockSpec(memory_space=pltpu.VMEM),
            pl.BlockSpec(memory_space=pltpu.VMEM),
        ],
        out_specs=pl.BlockSpec(memory_space=pltpu.VMEM),
        scratch_shapes=[
            pltpu.VMEM((2, M_PER, NH), BF16),
            pltpu.VMEM((2, M_PER, NH), BF16),
            pltpu.SemaphoreType.DMA((2,)),
            pltpu.SemaphoreType.DMA((2,)),
            pltpu.SemaphoreType.DMA((2,)),
            pltpu.SemaphoreType.DMA((2,)),
            pltpu.SemaphoreType.REGULAR,
            pltpu.SemaphoreType.REGULAR,
            pltpu.VMEM((1, 128), F32),
            pltpu.VMEM((3, 1, 128), F32),
            pltpu.SemaphoreType.DMA((3,)),
            pltpu.SemaphoreType.DMA((3,)),
        ],
        compiler_params=pltpu.CompilerParams(
            collective_id=0, vmem_limit_bytes=64 * 1024 * 1024
        ),
    )(x, w_mat)


# device time: 370363 ns/iter; 1.1588x vs baseline; 1.1588x over previous
import jax
import jax.numpy as jnp
from jax import lax
from jax.experimental import pallas as pl
from jax.experimental.pallas import tpu as pltpu

N_DEV = 8
M_PER = 512
N = 8192
NH = 4096
SC = 4
ST = NH // SC
CB = 512
F32 = jnp.float32
BF16 = jnp.bfloat16


def _ring_to_logical(ring_pos):
    return jnp.where(ring_pos < 4, ring_pos, 11 - ring_pos)


def kernel(x, w_mat):
    assert x.shape == (4096, 512) and w_mat.shape == (512, N)
    x = x.astype(BF16)
    w_mat = w_mat.astype(BF16)

    def body(
        x_ref,
        w_ref,
        out_ref,
        buf_r,
        buf_l,
        ssem_r,
        rsem_r,
        ssem_l,
        rsem_l,
        cred_r,
        cred_l,
        my_amax,
        rx_amax,
        ssem_ax,
        rsem_ax,
    ):
        my_log = lax.axis_index("i")
        pos = _ring_to_logical(my_log)
        right_log = _ring_to_logical((pos + 1) % N_DEV)
        left_log = _ring_to_logical((pos - 1) % N_DEV)

        barrier_sem = pltpu.get_barrier_semaphore()
        for nbr in (left_log, right_log):
            pl.semaphore_signal(
                barrier_sem,
                inc=1,
                device_id=(nbr,),
                device_id_type=pl.DeviceIdType.MESH,
            )
        pl.semaphore_wait(barrier_sem, 2)

        def dot_block(chunk_log, lo, hi):
            return jnp.dot(
                x_ref[pl.ds(chunk_log * M_PER, M_PER), :],
                w_ref[:, lo:hi],
                preferred_element_type=F32,
            )

        def seed_stripe(buf, sc, chunk_log, base):
            for b in range(sc * ST, (sc + 1) * ST, CB):
                buf[1, :, b : b + CB] = dot_block(
                    chunk_log, base + b, base + b + CB
                ).astype(BF16)

        def accum_stripe(buf, slot, sc, chunk_log, base):
            for b in range(sc * ST, (sc + 1) * ST, CB):
                buf[slot, :, b : b + CB] = (
                    buf[slot, :, b : b + CB].astype(F32)
                    + dot_block(chunk_log, base + b, base + b + CB)
                ).astype(BF16)

        def stripe_rdma(buf, src_slot, dst_slot, sc, ssem, rsem, dev):
            return pltpu.make_async_remote_copy(
                src_ref=buf.at[src_slot, :, pl.ds(sc * ST, ST)],
                dst_ref=buf.at[dst_slot, :, pl.ds(sc * ST, ST)],
                send_sem=ssem.at[dst_slot, sc],
                recv_sem=rsem.at[dst_slot, sc],
                device_id=(dev,),
                device_id_type=pl.DeviceIdType.MESH,
            )

        rdma_r = [[None] * SC for _ in range(N_DEV - 1)]
        rdma_l = [[None] * SC for _ in range(N_DEV - 1)]
        for s in range(N_DEV - 1):
            c_r = _ring_to_logical((pos - 1 - s) % N_DEV)
            c_l = _ring_to_logical((pos + 1 + s) % N_DEV)
            src = (s - 1) % 2
            dst = s % 2

            for sc in range(SC):
                if s == 0:
                    seed_stripe(buf_r, sc, c_r, 0)
                    seed_stripe(buf_l, sc, c_l, NH)
                else:
                    rdma_r[s - 1][sc].wait_recv()
                    accum_stripe(buf_r, src, sc, c_r, 0)
                    rdma_l[s - 1][sc].wait_recv()
                    accum_stripe(buf_l, src, sc, c_l, NH)
                    rdma_r[s - 1][sc].wait_send()
                    rdma_l[s - 1][sc].wait_send()
                    pl.semaphore_signal(
                        cred_r,
                        inc=1,
                        device_id=(left_log,),
                        device_id_type=pl.DeviceIdType.MESH,
                    )
                    pl.semaphore_signal(
                        cred_l,
                        inc=1,
                        device_id=(right_log,),
                        device_id_type=pl.DeviceIdType.MESH,
                    )
                    pl.semaphore_wait(cred_r, 1)
                    pl.semaphore_wait(cred_l, 1)

                rdma_r[s][sc] = stripe_rdma(
                    buf_r, src, dst, sc, ssem_r, rsem_r, right_log
                )
                rdma_r[s][sc].start()
                rdma_l[s][sc] = stripe_rdma(
                    buf_l, src, dst, sc, ssem_l, rsem_l, left_log
                )
                rdma_l[s][sc].start()

            if s == 0:
                for b in range(0, N, CB):
                    out_ref[:, b : b + CB] = dot_block(my_log, b, b + CB)

        amax = F32(0.0)
        for sc in range(SC):
            rdma_r[N_DEV - 2][sc].wait_recv()
            for b in range(sc * ST, (sc + 1) * ST, CB):
                blk = jnp.maximum(
                    out_ref[:, b : b + CB] + buf_r[0, :, b : b + CB].astype(F32),
                    0.0,
                )
                out_ref[:, b : b + CB] = blk
                amax = jnp.maximum(amax, jnp.max(blk))
            rdma_l[N_DEV - 2][sc].wait_recv()
            for b in range(NH + sc * ST, NH + (sc + 1) * ST, CB):
                blk = jnp.maximum(
                    out_ref[:, b : b + CB]
                    + buf_l[0, :, b - NH : b - NH + CB].astype(F32),
                    0.0,
                )
                out_ref[:, b : b + CB] = blk
                amax = jnp.maximum(amax, jnp.max(blk))
        for sc in range(SC):
            rdma_r[N_DEV - 2][sc].wait_send()
            rdma_l[N_DEV - 2][sc].wait_send()

        m_in = my_log % 4
        base = my_log - m_in
        partners = [
            base + jnp.bitwise_xor(m_in, 1),
            base + (3 - m_in),
            (my_log + 4) % N_DEV,
        ]
        my_amax[:, :] = jnp.full((1, 128), amax, F32)
        for h, pid in enumerate(partners):
            ex = pltpu.make_async_remote_copy(
                src_ref=my_amax,
                dst_ref=rx_amax.at[h],
                send_sem=ssem_ax.at[h],
                recv_sem=rsem_ax.at[h],
                device_id=(pid,),
                device_id_type=pl.DeviceIdType.MESH,
            )
            ex.start()
            ex.wait()
            my_amax[:, :] = jnp.maximum(my_amax[:, :], rx_amax[h, :, :])

        scale = my_amax[0, 0] / 127.0
        for b in range(0, N, CB):
            q = jnp.clip(
                jnp.round(out_ref[:, b : b + CB] / scale), -127.0, 127.0
            )
            out_ref[:, b : b + CB] = q * scale

    return pl.pallas_call(
        body,
        out_shape=jax.ShapeDtypeStruct((M_PER, N), F32),
        in_specs=[
            pl.BlockSpec(memory_space=pltpu.VMEM),
            pl.BlockSpec(memory_space=pltpu.VMEM),
        ],
        out_specs=pl.BlockSpec(memory_space=pltpu.VMEM),
        scratch_shapes=[
            pltpu.VMEM((2, M_PER, NH), BF16),
            pltpu.VMEM((2, M_PER, NH), BF16),
            pltpu.SemaphoreType.DMA((2, SC)),
            pltpu.SemaphoreType.DMA((2, SC)),
            pltpu.SemaphoreType.DMA((2, SC)),
            pltpu.SemaphoreType.DMA((2, SC)),
            pltpu.SemaphoreType.REGULAR,
            pltpu.SemaphoreType.REGULAR,
            pltpu.VMEM((1, 128), F32),
            pltpu.VMEM((3, 1, 128), F32),
            pltpu.SemaphoreType.DMA((3,)),
            pltpu.SemaphoreType.DMA((3,)),
        ],
        compiler_params=pltpu.CompilerParams(
            collective_id=0, vmem_limit_bytes=64 * 1024 * 1024
        ),
    )(x, w_mat)


# device time: 366001 ns/iter; 1.1726x vs baseline; 1.0119x over previous
import jax
import jax.numpy as jnp
from jax import lax
from jax.experimental import pallas as pl
from jax.experimental.pallas import tpu as pltpu

N_DEV = 8
M_PER = 512
N = 8192
NH = 4096
SC = 8
ST = NH // SC
CB = 512
F32 = jnp.float32
BF16 = jnp.bfloat16


def _ring_to_logical(ring_pos):
    return jnp.where(ring_pos < 4, ring_pos, 11 - ring_pos)


def kernel(x, w_mat):
    assert x.shape == (4096, 512) and w_mat.shape == (512, N)
    w_mat = w_mat.astype(BF16)

    def body(
        x_ref,
        w_ref,
        out_ref,
        buf_r,
        buf_l,
        ssem_r,
        rsem_r,
        ssem_l,
        rsem_l,
        cred_r,
        cred_l,
        my_amax,
        rx_amax,
        ssem_ax,
        rsem_ax,
    ):
        my_log = lax.axis_index("i")
        pos = _ring_to_logical(my_log)
        right_log = _ring_to_logical((pos + 1) % N_DEV)
        left_log = _ring_to_logical((pos - 1) % N_DEV)

        barrier_sem = pltpu.get_barrier_semaphore()
        for nbr in (left_log, right_log):
            pl.semaphore_signal(
                barrier_sem,
                inc=1,
                device_id=(nbr,),
                device_id_type=pl.DeviceIdType.MESH,
            )
        pl.semaphore_wait(barrier_sem, 2)

        def dot_block(chunk_log, lo, hi):
            return jnp.dot(
                x_ref[pl.ds(chunk_log * M_PER, M_PER), :].astype(BF16),
                w_ref[:, lo:hi],
                preferred_element_type=F32,
            )

        def seed_stripe(buf, sc, chunk_log, base):
            for b in range(sc * ST, (sc + 1) * ST, CB):
                buf[1, :, b : b + CB] = dot_block(
                    chunk_log, base + b, base + b + CB
                ).astype(BF16)

        def accum_stripe(buf, slot, sc, chunk_log, base):
            for b in range(sc * ST, (sc + 1) * ST, CB):
                buf[slot, :, b : b + CB] = (
                    buf[slot, :, b : b + CB].astype(F32)
                    + dot_block(chunk_log, base + b, base + b + CB)
                ).astype(BF16)

        def stripe_rdma(buf, src_slot, dst_slot, sc, ssem, rsem, dev):
            return pltpu.make_async_remote_copy(
                src_ref=buf.at[src_slot, :, pl.ds(sc * ST, ST)],
                dst_ref=buf.at[dst_slot, :, pl.ds(sc * ST, ST)],
                send_sem=ssem.at[dst_slot, sc],
                recv_sem=rsem.at[dst_slot, sc],
                device_id=(dev,),
                device_id_type=pl.DeviceIdType.MESH,
            )

        rdma_r = [[None] * SC for _ in range(N_DEV - 1)]
        rdma_l = [[None] * SC for _ in range(N_DEV - 1)]
        for s in range(N_DEV - 1):
            c_r = _ring_to_logical((pos - 1 - s) % N_DEV)
            c_l = _ring_to_logical((pos + 1 + s) % N_DEV)
            src = (s - 1) % 2
            dst = s % 2

            for sc in range(SC):
                if s == 0:
                    seed_stripe(buf_r, sc, c_r, 0)
                    seed_stripe(buf_l, sc, c_l, NH)
                else:
                    rdma_r[s - 1][sc].wait_recv()
                    accum_stripe(buf_r, src, sc, c_r, 0)
                    rdma_l[s - 1][sc].wait_recv()
                    accum_stripe(buf_l, src, sc, c_l, NH)
                    rdma_r[s - 1][sc].wait_send()
                    rdma_l[s - 1][sc].wait_send()
                    pl.semaphore_signal(
                        cred_r,
                        inc=1,
                        device_id=(left_log,),
                        device_id_type=pl.DeviceIdType.MESH,
                    )
                    pl.semaphore_signal(
                        cred_l,
                        inc=1,
                        device_id=(right_log,),
                        device_id_type=pl.DeviceIdType.MESH,
                    )
                    pl.semaphore_wait(cred_r, 1)
                    pl.semaphore_wait(cred_l, 1)

                rdma_r[s][sc] = stripe_rdma(
                    buf_r, src, dst, sc, ssem_r, rsem_r, right_log
                )
                rdma_r[s][sc].start()
                rdma_l[s][sc] = stripe_rdma(
                    buf_l, src, dst, sc, ssem_l, rsem_l, left_log
                )
                rdma_l[s][sc].start()

            if s == 0:
                for b in range(0, N, CB):
                    out_ref[:, b : b + CB] = dot_block(my_log, b, b + CB)

        amax = F32(0.0)
        for sc in range(SC):
            rdma_r[N_DEV - 2][sc].wait_recv()
            for b in range(sc * ST, (sc + 1) * ST, CB):
                blk = jnp.maximum(
                    out_ref[:, b : b + CB] + buf_r[0, :, b : b + CB].astype(F32),
                    0.0,
                )
                out_ref[:, b : b + CB] = blk
                amax = jnp.maximum(amax, jnp.max(blk))
            rdma_l[N_DEV - 2][sc].wait_recv()
            for b in range(NH + sc * ST, NH + (sc + 1) * ST, CB):
                blk = jnp.maximum(
                    out_ref[:, b : b + CB]
                    + buf_l[0, :, b - NH : b - NH + CB].astype(F32),
                    0.0,
                )
                out_ref[:, b : b + CB] = blk
                amax = jnp.maximum(amax, jnp.max(blk))
        for sc in range(SC):
            rdma_r[N_DEV - 2][sc].wait_send()
            rdma_l[N_DEV - 2][sc].wait_send()

        m_in = my_log % 4
        base = my_log - m_in
        partners = [
            base + jnp.bitwise_xor(m_in, 1),
            base + (3 - m_in),
            (my_log + 4) % N_DEV,
        ]
        my_amax[:, :] = jnp.full((1, 128), amax, F32)
        for h, pid in enumerate(partners):
            ex = pltpu.make_async_remote_copy(
                src_ref=my_amax,
                dst_ref=rx_amax.at[h],
                send_sem=ssem_ax.at[h],
                recv_sem=rsem_ax.at[h],
                device_id=(pid,),
                device_id_type=pl.DeviceIdType.MESH,
            )
            ex.start()
            ex.wait()
            my_amax[:, :] = jnp.maximum(my_amax[:, :], rx_amax[h, :, :])

        scale = my_amax[0, 0] / 127.0
        for b in range(0, N, CB):
            q = jnp.clip(
                jnp.round(out_ref[:, b : b + CB] / scale), -127.0, 127.0
            )
            out_ref[:, b : b + CB] = q * scale

    return pl.pallas_call(
        body,
        out_shape=jax.ShapeDtypeStruct((M_PER, N), F32),
        in_specs=[
            pl.BlockSpec(memory_space=pltpu.VMEM),
            pl.BlockSpec(memory_space=pltpu.VMEM),
        ],
        out_specs=pl.BlockSpec(memory_space=pltpu.VMEM),
        scratch_shapes=[
            pltpu.VMEM((2, M_PER, NH), BF16),
            pltpu.VMEM((2, M_PER, NH), BF16),
            pltpu.SemaphoreType.DMA((2, SC)),
            pltpu.SemaphoreType.DMA((2, SC)),
            pltpu.SemaphoreType.DMA((2, SC)),
            pltpu.SemaphoreType.DMA((2, SC)),
            pltpu.SemaphoreType.REGULAR,
            pltpu.SemaphoreType.REGULAR,
            pltpu.VMEM((1, 128), F32),
            pltpu.VMEM((3, 1, 128), F32),
            pltpu.SemaphoreType.DMA((3,)),
            pltpu.SemaphoreType.DMA((3,)),
        ],
        compiler_params=pltpu.CompilerParams(
            collective_id=0, vmem_limit_bytes=64 * 1024 * 1024
        ),
    )(x, w_mat)


# device time: 358894 ns/iter; 1.1958x vs baseline; 1.0198x over previous
import jax
import jax.numpy as jnp
from jax import lax
from jax.experimental import pallas as pl
from jax.experimental.pallas import tpu as pltpu

N_DEV = 8
M_PER = 512
N = 8192
NH = 4096
SC = 8
ST = NH // SC
CB = 512
F32 = jnp.float32
BF16 = jnp.bfloat16


def _ring_to_logical(ring_pos):
    return jnp.where(ring_pos < 4, ring_pos, 11 - ring_pos)


def kernel(x, w_mat):
    assert x.shape == (4096, 512) and w_mat.shape == (512, N)

    def body(
        x_ref,
        w_ref,
        out_ref,
        buf_r,
        buf_l,
        ssem_r,
        rsem_r,
        ssem_l,
        rsem_l,
        cred_r,
        cred_l,
        my_amax,
        rx_amax,
        ssem_ax,
        rsem_ax,
    ):
        my_log = lax.axis_index("i")
        pos = _ring_to_logical(my_log)
        right_log = _ring_to_logical((pos + 1) % N_DEV)
        left_log = _ring_to_logical((pos - 1) % N_DEV)

        barrier_sem = pltpu.get_barrier_semaphore()
        for nbr in (left_log, right_log):
            pl.semaphore_signal(
                barrier_sem,
                inc=1,
                device_id=(nbr,),
                device_id_type=pl.DeviceIdType.MESH,
            )
        pl.semaphore_wait(barrier_sem, 2)

        def dot_block(chunk_log, lo, hi):
            return jnp.dot(
                x_ref[pl.ds(chunk_log * M_PER, M_PER), :].astype(BF16),
                w_ref[:, lo:hi].astype(BF16),
                preferred_element_type=F32,
            )

        def seed_stripe(buf, sc, chunk_log, base):
            for b in range(sc * ST, (sc + 1) * ST, CB):
                buf[1, :, b : b + CB] = dot_block(
                    chunk_log, base + b, base + b + CB
                ).astype(BF16)

        def accum_stripe(buf, slot, sc, chunk_log, base):
            for b in range(sc * ST, (sc + 1) * ST, CB):
                buf[slot, :, b : b + CB] = (
                    buf[slot, :, b : b + CB].astype(F32)
                    + dot_block(chunk_log, base + b, base + b + CB)
                ).astype(BF16)

        def stripe_rdma(buf, src_slot, dst_slot, sc, ssem, rsem, dev):
            return pltpu.make_async_remote_copy(
                src_ref=buf.at[src_slot, :, pl.ds(sc * ST, ST)],
                dst_ref=buf.at[dst_slot, :, pl.ds(sc * ST, ST)],
                send_sem=ssem.at[dst_slot, sc],
                recv_sem=rsem.at[dst_slot, sc],
                device_id=(dev,),
                device_id_type=pl.DeviceIdType.MESH,
            )

        rdma_r = [[None] * SC for _ in range(N_DEV - 1)]
        rdma_l = [[None] * SC for _ in range(N_DEV - 1)]
        for s in range(N_DEV - 1):
            c_r = _ring_to_logical((pos - 1 - s) % N_DEV)
            c_l = _ring_to_logical((pos + 1 + s) % N_DEV)
            src = (s - 1) % 2
            dst = s % 2

            for sc in range(SC):
                if s == 0:
                    seed_stripe(buf_r, sc, c_r, 0)
                    seed_stripe(buf_l, sc, c_l, NH)
                else:
                    rdma_r[s - 1][sc].wait_recv()
                    accum_stripe(buf_r, src, sc, c_r, 0)
                    rdma_l[s - 1][sc].wait_recv()
                    accum_stripe(buf_l, src, sc, c_l, NH)
                    rdma_r[s - 1][sc].wait_send()
                    rdma_l[s - 1][sc].wait_send()
                    pl.semaphore_signal(
                        cred_r,
                        inc=1,
                        device_id=(left_log,),
                        device_id_type=pl.DeviceIdType.MESH,
                    )
                    pl.semaphore_signal(
                        cred_l,
                        inc=1,
                        device_id=(right_log,),
                        device_id_type=pl.DeviceIdType.MESH,
                    )
                    pl.semaphore_wait(cred_r, 1)
                    pl.semaphore_wait(cred_l, 1)

                rdma_r[s][sc] = stripe_rdma(
                    buf_r, src, dst, sc, ssem_r, rsem_r, right_log
                )
                rdma_r[s][sc].start()
                rdma_l[s][sc] = stripe_rdma(
                    buf_l, src, dst, sc, ssem_l, rsem_l, left_log
                )
                rdma_l[s][sc].start()

            if s == 0:
                for b in range(0, N, CB):
                    out_ref[:, b : b + CB] = dot_block(my_log, b, b + CB)

        amax = F32(0.0)
        for sc in range(SC):
            rdma_r[N_DEV - 2][sc].wait_recv()
            for b in range(sc * ST, (sc + 1) * ST, CB):
                blk = jnp.maximum(
                    out_ref[:, b : b + CB] + buf_r[0, :, b : b + CB].astype(F32),
                    0.0,
                )
                out_ref[:, b : b + CB] = blk
                amax = jnp.maximum(amax, jnp.max(blk))
            rdma_l[N_DEV - 2][sc].wait_recv()
            for b in range(NH + sc * ST, NH + (sc + 1) * ST, CB):
                blk = jnp.maximum(
                    out_ref[:, b : b + CB]
                    + buf_l[0, :, b - NH : b - NH + CB].astype(F32),
                    0.0,
                )
                out_ref[:, b : b + CB] = blk
                amax = jnp.maximum(amax, jnp.max(blk))
        for sc in range(SC):
            rdma_r[N_DEV - 2][sc].wait_send()
            rdma_l[N_DEV - 2][sc].wait_send()

        m_in = my_log % 4
        base = my_log - m_in
        partners = [
            base + jnp.bitwise_xor(m_in, 1),
            base + (3 - m_in),
            (my_log + 4) % N_DEV,
        ]
        my_amax[:, :] = jnp.full((1, 128), amax, F32)
        for h, pid in enumerate(partners):
            ex = pltpu.make_async_remote_copy(
                src_ref=my_amax,
                dst_ref=rx_amax.at[h],
                send_sem=ssem_ax.at[h],
                recv_sem=rsem_ax.at[h],
                device_id=(pid,),
                device_id_type=pl.DeviceIdType.MESH,
            )
            ex.start()
            ex.wait()
            my_amax[:, :] = jnp.maximum(my_amax[:, :], rx_amax[h, :, :])

        scale = my_amax[0, 0] / 127.0
        inv_scale = 127.0 / my_amax[0, 0]
        for b in range(0, N, CB):
            q = jnp.clip(
                jnp.round(out_ref[:, b : b + CB] * inv_scale), -127.0, 127.0
            )
            out_ref[:, b : b + CB] = q * scale

    return pl.pallas_call(
        body,
        out_shape=jax.ShapeDtypeStruct((M_PER, N), F32),
        in_specs=[
            pl.BlockSpec(memory_space=pltpu.VMEM),
            pl.BlockSpec(memory_space=pltpu.VMEM),
        ],
        out_specs=pl.BlockSpec(memory_space=pltpu.VMEM),
        scratch_shapes=[
            pltpu.VMEM((2, M_PER, NH), BF16),
            pltpu.VMEM((2, M_PER, NH), BF16),
            pltpu.SemaphoreType.DMA((2, SC)),
            pltpu.SemaphoreType.DMA((2, SC)),
            pltpu.SemaphoreType.DMA((2, SC)),
            pltpu.SemaphoreType.DMA((2, SC)),
            pltpu.SemaphoreType.REGULAR,
            pltpu.SemaphoreType.REGULAR,
            pltpu.VMEM((1, 128), F32),
            pltpu.VMEM((3, 1, 128), F32),
            pltpu.SemaphoreType.DMA((3,)),
            pltpu.SemaphoreType.DMA((3,)),
        ],
        compiler_params=pltpu.CompilerParams(
            collective_id=0, vmem_limit_bytes=64 * 1024 * 1024
        ),
    )(x, w_mat)
